# baseline (speedup 1.0000x reference)
import functools

import jax
import jax.numpy as jnp
from jax import lax
from jax.experimental import pallas as pl
from jax.experimental.pallas import tpu as pltpu

T, D, I, E, K = 2048, 1024, 512, 8, 2
BT = 256


def _dot_t(a, b):
    return lax.dot_general(a, b, (((1,), (1,)), ((), ())))


def _moe_body(x_ref, gate_w_ref, gp_ref, up_ref, dp_ref, sg_ref, su_ref,
              sd_ref, out_ref, logits_ref, ids_ref, acc_ref):
    g = pl.program_id(0)
    tb = pl.program_id(1)
    xb = x_ref[...]
    iota_e = lax.broadcasted_iota(jnp.int32, (BT, E), 1)

    logits = _dot_t(xb, gate_w_ref[...])
    m = jnp.max(logits, axis=1, keepdims=True)
    p = jnp.exp(logits - m)
    probs = p / jnp.sum(p, axis=1, keepdims=True)
    m1 = jnp.max(probs, axis=1, keepdims=True)
    a1 = jnp.min(jnp.where(probs == m1, iota_e, E), axis=1, keepdims=True)
    probs2 = jnp.where(iota_e == a1, -1.0, probs)
    m2 = jnp.max(probs2, axis=1, keepdims=True)
    a2 = jnp.min(jnp.where(probs2 == m2, iota_e, E), axis=1, keepdims=True)
    s = m1 + m2 + 1e-9
    comb = (jnp.where(iota_e == a1, m1 / s, 0.0)
            + jnp.where(iota_e == a2, m2 / s, 0.0))
    row = tb * BT

    @pl.when(g == 0)
    def _router_out():
        logits_ref[...] = logits
        ids_ref[...] = jnp.concatenate([a1, a2], axis=1)

    @pl.when(g == 0)
    def _shared():
        hg = _dot_t(xb, sg_ref[...])
        hu = _dot_t(xb, su_ref[...])
        h = hg / (1.0 + jnp.exp(-hg)) * hu
        acc_ref[pl.ds(row, BT), :] = _dot_t(h, sd_ref[...])

    @pl.when(g > 0)
    def _expert():
        hg = _dot_t(xb, gp_ref[0])
        hu = _dot_t(xb, up_ref[0])
        h = hg / (1.0 + jnp.exp(-hg)) * hu
        eo = _dot_t(h, dp_ref[0])
        w = jnp.sum(comb * (iota_e == (g - 1)).astype(jnp.float32),
                    axis=1, keepdims=True)
        acc_ref[pl.ds(row, BT), :] += w * eo

    @pl.when(g == E)
    def _emit():
        out_ref[...] = acc_ref[pl.ds(row, BT), :]


@jax.jit
def _moe_dense(x, gate_w, gp, up, dp, sg, su, sd):
    grid = (E + 1, T // BT)

    def _wmap(g, tb):
        return (jnp.maximum(g - 1, 0), 0, 0)

    return pl.pallas_call(
        _moe_body,
        grid=grid,
        in_specs=[
            pl.BlockSpec((BT, D), lambda g, tb: (tb, 0)),
            pl.BlockSpec((E, D), lambda g, tb: (0, 0)),
            pl.BlockSpec((1, I, D), _wmap),
            pl.BlockSpec((1, I, D), _wmap),
            pl.BlockSpec((1, D, I), _wmap),
            pl.BlockSpec((I, D), lambda g, tb: (0, 0)),
            pl.BlockSpec((I, D), lambda g, tb: (0, 0)),
            pl.BlockSpec((D, I), lambda g, tb: (0, 0)),
        ],
        out_specs=[
            pl.BlockSpec((BT, D), lambda g, tb: (jnp.where(g == E, tb, 0), 0)),
            pl.BlockSpec((BT, E),
                         lambda g, tb: (jnp.where(g == 0, tb, T // BT - 1), 0)),
            pl.BlockSpec((BT, K),
                         lambda g, tb: (jnp.where(g == 0, tb, T // BT - 1), 0)),
        ],
        out_shape=[
            jax.ShapeDtypeStruct((T, D), jnp.float32),
            jax.ShapeDtypeStruct((T, E), jnp.float32),
            jax.ShapeDtypeStruct((T, K), jnp.int32),
        ],
        scratch_shapes=[
            pltpu.VMEM((T, D), jnp.float32),
        ],
    )(x, gate_w, gp, up, dp, sg, su, sd)


def kernel(hidden_state, gate_w, gate_proj, up_proj, down_proj, shared_gate,
           shared_up, shared_down):
    Bv, Nv, Dv = hidden_state.shape
    x = hidden_state.reshape(Bv * Nv, Dv)
    out, logits, ids = _moe_dense(x, gate_w, gate_proj, up_proj, down_proj,
                                  shared_gate, shared_up, shared_down)
    return out.reshape(Bv, Nv, Dv), logits, ids
